# Initial kernel scaffold; baseline (speedup 1.0000x reference)
import jax
import jax.numpy as jnp
from jax.experimental import pallas as pl


def kernel(x, edge_index, edge_attr, timestep, params):
    raise NotImplementedError("write your pallas kernel here")


# speedup vs baseline: 2.2611x; 2.2611x over previous
import functools

import jax
import jax.numpy as jnp
import numpy as np
from jax import lax
from jax.experimental import pallas as pl
from jax.experimental.pallas import tpu as pltpu
from jax.experimental.pallas import tpu_sc as plsc

N = 50000
E = 800000
HID = 64
TDIM = 32
MAXPOS = 1000
NMP = 5

NC = 2
NS = 16
CHUNK = 128
EPAD = 802816
CPT = EPAD // NS // CHUNK
NHALF = N // 2
NGARB = 600
NLOC = NHALF + NGARB
ZSTRIPE = NLOC // NS
OSTRIPE = 1560


def _ln(h, g, b):
    m = jnp.mean(h, axis=-1, keepdims=True)
    v = jnp.mean((h - m) ** 2, axis=-1, keepdims=True)
    return (h - m) / jnp.sqrt(v + 1e-5) * g + b


def _dot(a, b):
    return jax.lax.dot_general(a, b, (((1,), (0,)), ((), ())),
                               preferred_element_type=jnp.float32)


def _dot_hi(a, b):
    return jax.lax.dot_general(a, b, (((1,), (0,)), ((), ())),
                               precision=jax.lax.Precision.HIGHEST,
                               preferred_element_type=jnp.float32)


def _bf16r(x):
    return x.astype(jnp.bfloat16).astype(jnp.float32)


def _enc2_body(h_ref, w1, b1, g1, be1, w2, b2, g2, be2, o_ref):
    t = jnp.maximum(_dot(h_ref[...], w1[...]) + b1[...], 0.0)
    t = _ln(t, g1[...], be1[...])
    t = jnp.maximum(_dot(t, w2[...]) + b2[...], 0.0)
    o_ref[...] = _ln(t, g2[...], be2[...])


def _mlp2(h, l1, l2, blk):
    n, din = h.shape
    grid = n // blk
    wspec = lambda shape: pl.BlockSpec(shape, lambda i: (0, 0))
    return pl.pallas_call(
        _enc2_body,
        grid=(grid,),
        in_specs=[
            pl.BlockSpec((blk, din), lambda i: (i, 0)),
            wspec((din, HID)), wspec((1, HID)), wspec((1, HID)), wspec((1, HID)),
            wspec((HID, HID)), wspec((1, HID)), wspec((1, HID)), wspec((1, HID)),
        ],
        out_specs=pl.BlockSpec((blk, HID), lambda i: (i, 0)),
        out_shape=jax.ShapeDtypeStruct((n, HID), jnp.float32),
    )(h, l1["Wt"], l1["b"], l1["g"], l1["beta"],
      l2["Wt"], l2["b"], l2["g"], l2["beta"])


def _comb_body(nd_ref, g_ref, ea_ref, at, bt, w1n, w1a, b1, g1, be1,
               w2, b2, g2, be2, wn, lng, lnb, o_ref):
    nd = nd_ref[...]
    agg = _dot_hi(g_ref[...], at[...]) + _dot_hi(ea_ref[...], bt[...])
    t = jnp.maximum(_dot(nd, w1n[...]) + _dot(agg, w1a[...]) + b1[...], 0.0)
    t = _ln(t, g1[...], be1[...])
    t = jnp.maximum(_dot(t, w2[...]) + b2[...], 0.0)
    t = _ln(t, g2[...], be2[...])
    o_ref[...] = _ln(_dot(nd, wn[...]) + t, lng[...], lnb[...])


def _combine(nodes, g, ea, wr, blk):
    grid = N // blk
    wspec = lambda: pl.BlockSpec((HID, HID), lambda i: (0, 0))
    vspec = lambda: pl.BlockSpec((1, HID), lambda i: (0, 0))
    xspec = pl.BlockSpec((blk, HID), lambda i: (i, 0))
    return pl.pallas_call(
        _comb_body,
        grid=(grid,),
        in_specs=[xspec, xspec, xspec,
                  wspec(), wspec(), wspec(), wspec(), vspec(), vspec(), vspec(),
                  wspec(), vspec(), vspec(), vspec(), wspec(), vspec(), vspec()],
        out_specs=xspec,
        out_shape=jax.ShapeDtypeStruct((N, HID), jnp.float32),
    )(nodes, g, ea, wr["At"], wr["Bt"], wr["W1n"], wr["W1a"], wr["b1"],
      wr["g1"], wr["be1"], wr["W2t"], wr["b2"], wr["g2"], wr["be2"],
      wr["Wnt"], wr["lng"], wr["lnb"])


def _dec_body(nd_ref, wd, bd, gd, bed, wo, bo, o_ref):
    t = jnp.maximum(_dot(nd_ref[...], wd[...]) + bd[...], 0.0)
    t = _ln(t, gd[...], bed[...])
    l = _dot(t, wo[...]) + bo[...]
    l0 = l[:, 0:1]
    l1 = l[:, 1:2]
    m = jnp.maximum(l0, l1)
    lse = m + jnp.log(jnp.exp(l0 - m) + jnp.exp(l1 - m))
    o_ref[...] = l[:, 0:2] - lse


def _decoder(nodes, wd, blk):
    grid = N // blk
    return pl.pallas_call(
        _dec_body,
        grid=(grid,),
        in_specs=[
            pl.BlockSpec((blk, HID), lambda i: (i, 0)),
            pl.BlockSpec((HID, HID), lambda i: (0, 0)),
            pl.BlockSpec((1, HID), lambda i: (0, 0)),
            pl.BlockSpec((1, HID), lambda i: (0, 0)),
            pl.BlockSpec((1, HID), lambda i: (0, 0)),
            pl.BlockSpec((HID, 8), lambda i: (0, 0)),
            pl.BlockSpec((1, 8), lambda i: (0, 0)),
        ],
        out_specs=pl.BlockSpec((blk, 2), lambda i: (i, 0)),
        out_shape=jax.ShapeDtypeStruct((N, 2), jnp.float32),
    )(nodes, wd["Wdt"], wd["bd"], wd["gd"], wd["bed"], wd["Wot"], wd["bo"])


def _segsum_body(table, senders, recv2, zeros, out, sidx, ridx, rows, acc, sem):
    cid = lax.axis_index("c")
    sid = lax.axis_index("s")
    zb = sid * ZSTRIPE
    pltpu.sync_copy(zeros.at[pl.ds(zb, ZSTRIPE)], acc.at[pl.ds(zb, ZSTRIPE)])
    plsc.subcore_barrier()

    def chunk_body(j, _):
        eb = (sid * CPT + j) * CHUNK
        eb = pl.multiple_of(eb, CHUNK)
        pltpu.sync_copy(senders.at[pl.ds(eb, CHUNK)], sidx)
        pltpu.sync_copy(recv2.at[cid, pl.ds(eb, CHUNK)], ridx)
        pltpu.async_copy(table.at[sidx], rows, sem).wait()
        pltpu.sync_copy(rows, acc.at[ridx], add=True)
        return 0

    lax.fori_loop(0, CPT, chunk_body, 0)
    plsc.subcore_barrier()
    nbase = cid * NHALF
    ob = sid * OSTRIPE
    pltpu.sync_copy(acc.at[pl.ds(ob, OSTRIPE)], out.at[pl.ds(nbase + ob, OSTRIPE)])

    @pl.when(sid == 0)
    def _():
        rb = NS * OSTRIPE
        pltpu.sync_copy(acc.at[pl.ds(rb, NHALF - NS * OSTRIPE)],
                        out.at[pl.ds(nbase + rb, NHALF - NS * OSTRIPE)])


def _segsum(table, senders, recv2, zeros):
    return pl.kernel(
        _segsum_body,
        out_type=jax.ShapeDtypeStruct((N, HID), jnp.float32),
        mesh=plsc.VectorSubcoreMesh(core_axis_name="c", subcore_axis_name="s"),
        scratch_types=[
            pltpu.VMEM((CHUNK,), jnp.int32),
            pltpu.VMEM((CHUNK,), jnp.int32),
            pltpu.VMEM((CHUNK, HID), jnp.float32),
            pltpu.VMEM_SHARED((NLOC, HID), jnp.float32),
            pltpu.SemaphoreType.DMA,
        ],
        compiler_params=pltpu.CompilerParams(use_tc_tiling_on_sc=False),
    )(table, senders, recv2, zeros)


def _prep_layer(l, din):
    W = l["W"]
    Wt = jnp.zeros((din, HID), jnp.float32).at[: W.shape[1], :].set(W.T)
    return {"Wt": Wt, "b": l["b"][None, :], "g": l["g"][None, :],
            "beta": l["beta"][None, :]}


def kernel(x, edge_index, edge_attr, timestep, params):
    pos = timestep.astype(jnp.float32)
    div = jnp.exp(jnp.arange(0, TDIM, 2, dtype=jnp.float32)
                  * (-np.log(MAXPOS) / TDIM))
    pe = jnp.zeros((pos.shape[0], TDIM), jnp.float32)
    pe = pe.at[:, 0::2].set(jnp.sin(pos[:, None] * div))
    pe = pe.at[:, 1::2].set(jnp.cos(pos[:, None] * div))

    h0 = jnp.concatenate(
        [x, jnp.broadcast_to(pe, (N, TDIM)), jnp.zeros((N, 30), jnp.float32)],
        axis=-1)

    ne = params["node_enc"]
    nodes = _mlp2(h0, _prep_layer(ne[0], HID), _prep_layer(ne[1], HID), 2000)

    ee = params["edge_enc"]
    ea8 = jnp.concatenate([edge_attr, jnp.zeros((E, 4), jnp.float32)], axis=-1)
    edges = _mlp2(ea8, _prep_layer(ee[0], 8), _prep_layer(ee[1], HID), 2000)

    senders = edge_index[0]
    receivers = edge_index[1]
    pad = EPAD - E
    send_pad = jnp.concatenate(
        [senders, (jnp.arange(pad, dtype=jnp.int32) * 17) % N])
    eiota_pad = jnp.concatenate(
        [jnp.arange(E, dtype=jnp.int32), jnp.zeros((pad,), jnp.int32)])
    recv_pad = jnp.concatenate(
        [receivers, jnp.full((pad,), N, jnp.int32)])
    garb = NHALF + jnp.bitwise_and(recv_pad, 511)
    loc0 = jnp.where((recv_pad >= 0) & (recv_pad < NHALF), recv_pad, garb)
    r1 = recv_pad - NHALF
    loc1 = jnp.where((r1 >= 0) & (r1 < NHALF), r1, garb)
    recv2 = jnp.stack([loc0, loc1])

    zeros_acc = jnp.zeros((NLOC, HID), jnp.float32)

    ea = _segsum(_bf16r(edges), eiota_pad, recv2, zeros_acc)

    rounds = []
    for lp in params["mp"]:
        Wm = lp["W_msg"]
        W1 = lp["node_mlp"][0]["W"]
        rounds.append({
            "At": _bf16r(Wm[:, :HID].T), "Bt": _bf16r(Wm[:, HID:].T),
            "W1n": W1[:, :HID].T, "W1a": W1[:, HID:].T,
            "b1": lp["node_mlp"][0]["b"][None, :],
            "g1": lp["node_mlp"][0]["g"][None, :],
            "be1": lp["node_mlp"][0]["beta"][None, :],
            "W2t": lp["node_mlp"][1]["W"].T,
            "b2": lp["node_mlp"][1]["b"][None, :],
            "g2": lp["node_mlp"][1]["g"][None, :],
            "be2": lp["node_mlp"][1]["beta"][None, :],
            "Wnt": lp["W_node"].T,
            "lng": lp["ln_g"][None, :], "lnb": lp["ln_b"][None, :],
        })

    for wr in rounds:
        g = _segsum(_bf16r(nodes), send_pad, recv2, zeros_acc)
        nodes = _combine(nodes, g, ea, wr, 2000)

    dh = params["dec_hidden"][0]
    do = params["dec_out"]
    wd = {"Wdt": dh["W"].T, "bd": dh["b"][None, :], "gd": dh["g"][None, :],
          "bed": dh["beta"][None, :],
          "Wot": jnp.concatenate(
              [do["W"].T, jnp.zeros((HID, 6), jnp.float32)], axis=-1),
          "bo": jnp.concatenate(
              [do["b"], jnp.full((6,), -1e30, jnp.float32)])[None, :]}
    return _decoder(nodes, wd, 2000)


# speedup vs baseline: 3.5350x; 1.5634x over previous
import functools

import jax
import jax.numpy as jnp
import numpy as np
from jax import lax
from jax.experimental import pallas as pl
from jax.experimental.pallas import tpu as pltpu
from jax.experimental.pallas import tpu_sc as plsc

N = 50000
E = 800000
HID = 64
TDIM = 32
MAXPOS = 1000
NMP = 5

NC = 2
NS = 16
CHUNK = 128
EPAD = 802816
CPT = EPAD // NS // CHUNK
GB = 28
NGRP = CPT // GB
NHALF = N // 2
NGARB = 600
NLOC = NHALF + NGARB
ZSTRIPE = NLOC // NS
OSTRIPE = 1560


def _ln(h, g, b):
    m = jnp.mean(h, axis=-1, keepdims=True)
    v = jnp.mean((h - m) ** 2, axis=-1, keepdims=True)
    return (h - m) / jnp.sqrt(v + 1e-5) * g + b


def _dot(a, b):
    return jax.lax.dot_general(a, b, (((1,), (0,)), ((), ())),
                               preferred_element_type=jnp.float32)


def _dot_hi(a, b):
    return jax.lax.dot_general(a, b, (((1,), (0,)), ((), ())),
                               precision=jax.lax.Precision.HIGHEST,
                               preferred_element_type=jnp.float32)


def _bf16r(x):
    return x.astype(jnp.bfloat16).astype(jnp.float32)


def _enc2_body(h_ref, w1, b1, g1, be1, w2, b2, g2, be2, o_ref):
    t = jnp.maximum(_dot(h_ref[...], w1[...]) + b1[...], 0.0)
    t = _ln(t, g1[...], be1[...])
    t = jnp.maximum(_dot(t, w2[...]) + b2[...], 0.0)
    o_ref[...] = _ln(t, g2[...], be2[...])


def _mlp2(h, l1, l2, blk):
    n, din = h.shape
    grid = n // blk
    wspec = lambda shape: pl.BlockSpec(shape, lambda i: (0, 0))
    return pl.pallas_call(
        _enc2_body,
        grid=(grid,),
        in_specs=[
            pl.BlockSpec((blk, din), lambda i: (i, 0)),
            wspec((din, HID)), wspec((1, HID)), wspec((1, HID)), wspec((1, HID)),
            wspec((HID, HID)), wspec((1, HID)), wspec((1, HID)), wspec((1, HID)),
        ],
        out_specs=pl.BlockSpec((blk, HID), lambda i: (i, 0)),
        out_shape=jax.ShapeDtypeStruct((n, HID), jnp.float32),
    )(h, l1["Wt"], l1["b"], l1["g"], l1["beta"],
      l2["Wt"], l2["b"], l2["g"], l2["beta"])


def _comb_body(nd_ref, g_ref, ea_ref, at, bt, w1n, w1a, b1, g1, be1,
               w2, b2, g2, be2, wn, lng, lnb, o_ref):
    nd = nd_ref[...]
    agg = _dot_hi(g_ref[...], at[...]) + _dot_hi(ea_ref[...], bt[...])
    t = jnp.maximum(_dot(nd, w1n[...]) + _dot(agg, w1a[...]) + b1[...], 0.0)
    t = _ln(t, g1[...], be1[...])
    t = jnp.maximum(_dot(t, w2[...]) + b2[...], 0.0)
    t = _ln(t, g2[...], be2[...])
    o_ref[...] = _ln(_dot(nd, wn[...]) + t, lng[...], lnb[...])


def _combine(nodes, g, ea, wr, blk):
    grid = N // blk
    wspec = lambda: pl.BlockSpec((HID, HID), lambda i: (0, 0))
    vspec = lambda: pl.BlockSpec((1, HID), lambda i: (0, 0))
    xspec = pl.BlockSpec((blk, HID), lambda i: (i, 0))
    return pl.pallas_call(
        _comb_body,
        grid=(grid,),
        in_specs=[xspec, xspec, xspec,
                  wspec(), wspec(), wspec(), wspec(), vspec(), vspec(), vspec(),
                  wspec(), vspec(), vspec(), vspec(), wspec(), vspec(), vspec()],
        out_specs=xspec,
        out_shape=jax.ShapeDtypeStruct((N, HID), jnp.float32),
    )(nodes, g, ea, wr["At"], wr["Bt"], wr["W1n"], wr["W1a"], wr["b1"],
      wr["g1"], wr["be1"], wr["W2t"], wr["b2"], wr["g2"], wr["be2"],
      wr["Wnt"], wr["lng"], wr["lnb"])


def _dec_body(nd_ref, wd, bd, gd, bed, wo, bo, o_ref):
    t = jnp.maximum(_dot(nd_ref[...], wd[...]) + bd[...], 0.0)
    t = _ln(t, gd[...], bed[...])
    l = _dot(t, wo[...]) + bo[...]
    l0 = l[:, 0:1]
    l1 = l[:, 1:2]
    m = jnp.maximum(l0, l1)
    lse = m + jnp.log(jnp.exp(l0 - m) + jnp.exp(l1 - m))
    o_ref[...] = l[:, 0:2] - lse


def _decoder(nodes, wd, blk):
    grid = N // blk
    return pl.pallas_call(
        _dec_body,
        grid=(grid,),
        in_specs=[
            pl.BlockSpec((blk, HID), lambda i: (i, 0)),
            pl.BlockSpec((HID, HID), lambda i: (0, 0)),
            pl.BlockSpec((1, HID), lambda i: (0, 0)),
            pl.BlockSpec((1, HID), lambda i: (0, 0)),
            pl.BlockSpec((1, HID), lambda i: (0, 0)),
            pl.BlockSpec((HID, 8), lambda i: (0, 0)),
            pl.BlockSpec((1, 8), lambda i: (0, 0)),
        ],
        out_specs=pl.BlockSpec((blk, 2), lambda i: (i, 0)),
        out_shape=jax.ShapeDtypeStruct((N, 2), jnp.float32),
    )(nodes, wd["Wdt"], wd["bd"], wd["gd"], wd["bed"], wd["Wot"], wd["bo"])


def _segsum_body(table, senders, recv2, zeros, out, sidx, ridx, rows0, rows1,
                 acc, sem0, sem1):
    cid = lax.axis_index("c")
    sid = lax.axis_index("s")
    zb = sid * ZSTRIPE
    pltpu.sync_copy(zeros.at[pl.ds(zb, ZSTRIPE)], acc.at[pl.ds(zb, ZSTRIPE)])
    plsc.subcore_barrier()

    def group_body(grp, _):
        pltpu.sync_copy(senders.at[sid, pl.ds(grp * GB, GB)], sidx)
        pltpu.sync_copy(recv2.at[cid, sid, pl.ds(grp * GB, GB)], ridx)

        pltpu.async_copy(table.at[sidx.at[0]], rows0, sem0)
        pltpu.async_copy(table.at[sidx.at[1]], rows1, sem1)

        def pair_body(i, _):
            g = 2 * i
            pltpu.make_async_copy(table.at[sidx.at[g]], rows0, sem0).wait()
            pltpu.sync_copy(rows0, acc.at[ridx.at[g]], add=True)

            @pl.when(g + 2 < GB)
            def _():
                pltpu.async_copy(table.at[sidx.at[g + 2]], rows0, sem0)

            pltpu.make_async_copy(table.at[sidx.at[g + 1]], rows1, sem1).wait()
            pltpu.sync_copy(rows1, acc.at[ridx.at[g + 1]], add=True)

            @pl.when(g + 3 < GB)
            def _():
                pltpu.async_copy(table.at[sidx.at[g + 3]], rows1, sem1)

            return 0

        lax.fori_loop(0, GB // 2, pair_body, 0)
        return 0

    lax.fori_loop(0, NGRP, group_body, 0)
    plsc.subcore_barrier()
    nbase = cid * NHALF
    ob = sid * OSTRIPE
    pltpu.sync_copy(acc.at[pl.ds(ob, OSTRIPE)], out.at[pl.ds(nbase + ob, OSTRIPE)])

    @pl.when(sid == 0)
    def _():
        rb = NS * OSTRIPE
        pltpu.sync_copy(acc.at[pl.ds(rb, NHALF - NS * OSTRIPE)],
                        out.at[pl.ds(nbase + rb, NHALF - NS * OSTRIPE)])


def _segsum(table, senders, recv2, zeros):
    return pl.kernel(
        _segsum_body,
        out_type=jax.ShapeDtypeStruct((N, HID), jnp.float32),
        mesh=plsc.VectorSubcoreMesh(core_axis_name="c", subcore_axis_name="s"),
        scratch_types=[
            pltpu.VMEM((GB, CHUNK), jnp.int32),
            pltpu.VMEM((GB, CHUNK), jnp.int32),
            pltpu.VMEM((CHUNK, HID), jnp.float32),
            pltpu.VMEM((CHUNK, HID), jnp.float32),
            pltpu.VMEM_SHARED((NLOC, HID), jnp.float32),
            pltpu.SemaphoreType.DMA,
            pltpu.SemaphoreType.DMA,
        ],
        compiler_params=pltpu.CompilerParams(use_tc_tiling_on_sc=False),
    )(table, senders, recv2, zeros)


def _prep_layer(l, din):
    W = l["W"]
    Wt = jnp.zeros((din, HID), jnp.float32).at[: W.shape[1], :].set(W.T)
    return {"Wt": Wt, "b": l["b"][None, :], "g": l["g"][None, :],
            "beta": l["beta"][None, :]}


def kernel(x, edge_index, edge_attr, timestep, params):
    pos = timestep.astype(jnp.float32)
    div = jnp.exp(jnp.arange(0, TDIM, 2, dtype=jnp.float32)
                  * (-np.log(MAXPOS) / TDIM))
    pe = jnp.zeros((pos.shape[0], TDIM), jnp.float32)
    pe = pe.at[:, 0::2].set(jnp.sin(pos[:, None] * div))
    pe = pe.at[:, 1::2].set(jnp.cos(pos[:, None] * div))

    h0 = jnp.concatenate(
        [x, jnp.broadcast_to(pe, (N, TDIM)), jnp.zeros((N, 30), jnp.float32)],
        axis=-1)

    ne = params["node_enc"]
    nodes = _mlp2(h0, _prep_layer(ne[0], HID), _prep_layer(ne[1], HID), 2000)

    ee = params["edge_enc"]
    ea8 = jnp.concatenate([edge_attr, jnp.zeros((E, 4), jnp.float32)], axis=-1)
    edges = _mlp2(ea8, _prep_layer(ee[0], 8), _prep_layer(ee[1], HID), 2000)

    senders = edge_index[0]
    receivers = edge_index[1]
    pad = EPAD - E
    send_pad = jnp.concatenate(
        [senders, (jnp.arange(pad, dtype=jnp.int32) * 17) % N])
    eiota_pad = jnp.concatenate(
        [jnp.arange(E, dtype=jnp.int32), jnp.zeros((pad,), jnp.int32)])
    recv_pad = jnp.concatenate(
        [receivers, jnp.full((pad,), N, jnp.int32)])
    garb = NHALF + jnp.bitwise_and(recv_pad, 511)
    loc0 = jnp.where((recv_pad >= 0) & (recv_pad < NHALF), recv_pad, garb)
    r1 = recv_pad - NHALF
    loc1 = jnp.where((r1 >= 0) & (r1 < NHALF), r1, garb)
    recv2 = jnp.stack([loc0, loc1]).reshape(2, NS, CPT, CHUNK)
    send_pad = send_pad.reshape(NS, CPT, CHUNK)
    eiota_pad = eiota_pad.reshape(NS, CPT, CHUNK)

    zeros_acc = jnp.zeros((NLOC, HID), jnp.float32)

    ea = _segsum(_bf16r(edges), eiota_pad, recv2, zeros_acc)

    rounds = []
    for lp in params["mp"]:
        Wm = lp["W_msg"]
        W1 = lp["node_mlp"][0]["W"]
        rounds.append({
            "At": _bf16r(Wm[:, :HID].T), "Bt": _bf16r(Wm[:, HID:].T),
            "W1n": W1[:, :HID].T, "W1a": W1[:, HID:].T,
            "b1": lp["node_mlp"][0]["b"][None, :],
            "g1": lp["node_mlp"][0]["g"][None, :],
            "be1": lp["node_mlp"][0]["beta"][None, :],
            "W2t": lp["node_mlp"][1]["W"].T,
            "b2": lp["node_mlp"][1]["b"][None, :],
            "g2": lp["node_mlp"][1]["g"][None, :],
            "be2": lp["node_mlp"][1]["beta"][None, :],
            "Wnt": lp["W_node"].T,
            "lng": lp["ln_g"][None, :], "lnb": lp["ln_b"][None, :],
        })

    for wr in rounds:
        g = _segsum(_bf16r(nodes), send_pad, recv2, zeros_acc)
        nodes = _combine(nodes, g, ea, wr, 2000)

    dh = params["dec_hidden"][0]
    do = params["dec_out"]
    wd = {"Wdt": dh["W"].T, "bd": dh["b"][None, :], "gd": dh["g"][None, :],
          "bed": dh["beta"][None, :],
          "Wot": jnp.concatenate(
              [do["W"].T, jnp.zeros((HID, 6), jnp.float32)], axis=-1),
          "bo": jnp.concatenate(
              [do["b"], jnp.full((6,), -1e30, jnp.float32)])[None, :]}
    return _decoder(nodes, wd, 2000)
